# Initial kernel scaffold; baseline (speedup 1.0000x reference)
import jax
import jax.numpy as jnp
from jax.experimental import pallas as pl


def kernel(inputs, targets_table, context_table):
    raise NotImplementedError("write your pallas kernel here")


# speedup vs baseline: 2.9751x; 2.9751x over previous
import functools

import jax
import jax.numpy as jnp
from jax import lax
from jax.experimental import pallas as pl
from jax.experimental.pallas import tpu as pltpu
from jax.experimental.pallas import tpu_sc as plsc

_NC = 2
_NS = 16
_NW = _NC * _NS
_CHUNK = 128


@functools.partial(jax.jit, static_argnums=(2, 3))
def _sc_gather(idx, table, n_chunks, embd):
    total = _NW * n_chunks * _CHUNK
    mesh = plsc.VectorSubcoreMesh(core_axis_name="c", subcore_axis_name="s")

    @functools.partial(
        pl.kernel,
        mesh=mesh,
        out_type=jax.ShapeDtypeStruct((total, embd), jnp.float32),
        scratch_types=[
            pltpu.VMEM((n_chunks, _CHUNK), jnp.int32),
            pltpu.VMEM((_CHUNK, embd), jnp.float32),
            pltpu.VMEM((_CHUNK, embd), jnp.float32),
            pltpu.SemaphoreType.DMA,
            pltpu.SemaphoreType.DMA,
        ],
    )
    def k(idx_hbm, table_hbm, out_hbm, idx_v, buf0, buf1, sem0, sem1):
        wid = lax.axis_index("s") * _NC + lax.axis_index("c")
        base = wid * (n_chunks * _CHUNK)
        pltpu.sync_copy(idx_hbm.at[wid], idx_v)

        def body(j, _):
            pltpu.async_copy(table_hbm.at[idx_v.at[j]], buf0, sem0).wait()
            pltpu.sync_copy(buf0, out_hbm.at[pl.ds(base + j * _CHUNK, _CHUNK)])
            return 0

        lax.fori_loop(0, n_chunks, body, 0)

    return k(idx, table)


def kernel(inputs, targets_table, context_table):
    b, h = inputs.shape
    v, d = targets_table.shape
    total = b * h
    assert total % (_NW * _CHUNK) == 0
    n_chunks = total // (_NW * _CHUNK)
    idx = inputs.reshape(_NW, n_chunks, _CHUNK).astype(jnp.int32)
    out = _sc_gather(idx, targets_table, n_chunks, d)
    return out.reshape(b, h, d)


# speedup vs baseline: 3.3208x; 1.1162x over previous
import functools

import jax
import jax.numpy as jnp
from jax import lax
from jax.experimental import pallas as pl
from jax.experimental.pallas import tpu as pltpu
from jax.experimental.pallas import tpu_sc as plsc

_NC = 2
_NS = 16
_NW = _NC * _NS
_CHUNK = 128


@functools.partial(jax.jit, static_argnums=(2, 3))
def _sc_gather(idx, table, n_chunks, embd):
    total = _NW * n_chunks * _CHUNK
    mesh = plsc.VectorSubcoreMesh(core_axis_name="c", subcore_axis_name="s")

    @functools.partial(
        pl.kernel,
        mesh=mesh,
        out_type=jax.ShapeDtypeStruct((total, embd), jnp.float32),
        scratch_types=[
            pltpu.VMEM((n_chunks, _CHUNK), jnp.int32),
            pltpu.VMEM((_CHUNK, embd), jnp.float32),
            pltpu.VMEM((_CHUNK, embd), jnp.float32),
            pltpu.SemaphoreType.DMA,
            pltpu.SemaphoreType.DMA,
        ],
    )
    def k(idx_hbm, table_hbm, out_hbm, idx_v, buf0, buf1, sem0, sem1):
        wid = lax.axis_index("s") * _NC + lax.axis_index("c")
        base = wid * (n_chunks * _CHUNK)
        pltpu.sync_copy(idx_hbm.at[wid], idx_v)

        def gather(j, buf, sem):
            pltpu.async_copy(table_hbm.at[idx_v.at[j]], buf, sem)

        def wait_gather(j, buf, sem):
            pltpu.make_async_copy(table_hbm.at[idx_v.at[j]], buf, sem).wait()

        def writeback(j, buf):
            pltpu.sync_copy(buf, out_hbm.at[pl.ds(base + j * _CHUNK, _CHUNK)])

        gather(0, buf0, sem0)

        def body(i, _):
            j0 = 2 * i
            gather(j0 + 1, buf1, sem1)
            wait_gather(j0, buf0, sem0)
            writeback(j0, buf0)

            @pl.when(j0 + 2 < n_chunks)
            def _():
                gather(j0 + 2, buf0, sem0)

            wait_gather(j0 + 1, buf1, sem1)
            writeback(j0 + 1, buf1)
            return 0

        lax.fori_loop(0, n_chunks // 2, body, 0)

    return k(idx, table)


def kernel(inputs, targets_table, context_table):
    b, h = inputs.shape
    v, d = targets_table.shape
    total = b * h
    assert total % (_NW * _CHUNK) == 0
    n_chunks = total // (_NW * _CHUNK)
    idx = inputs.reshape(_NW, n_chunks, _CHUNK).astype(jnp.int32)
    out = _sc_gather(idx, targets_table, n_chunks, d)
    return out.reshape(b, h, d)


# speedup vs baseline: 3.3354x; 1.0044x over previous
import functools

import jax
import jax.numpy as jnp
from jax import lax
from jax.experimental import pallas as pl
from jax.experimental.pallas import tpu as pltpu
from jax.experimental.pallas import tpu_sc as plsc

_NC = 2
_NS = 16
_NW = _NC * _NS
_CHUNK = 128
_NBUF = 5


@functools.partial(jax.jit, static_argnums=(2, 3))
def _sc_gather(idx, table, n_chunks, embd):
    total = _NW * n_chunks * _CHUNK
    mesh = plsc.VectorSubcoreMesh(core_axis_name="c", subcore_axis_name="s")

    @functools.partial(
        pl.kernel,
        mesh=mesh,
        out_type=jax.ShapeDtypeStruct((total, embd), jnp.float32),
        scratch_types=[
            pltpu.VMEM((n_chunks, _CHUNK), jnp.int32),
        ]
        + [pltpu.VMEM((_CHUNK, embd), jnp.float32) for _ in range(_NBUF)]
        + [pltpu.SemaphoreType.DMA for _ in range(2 * _NBUF)],
    )
    def k(idx_hbm, table_hbm, out_hbm, idx_v, *bufs_sems):
        bufs = bufs_sems[:_NBUF]
        gsems = bufs_sems[_NBUF : 2 * _NBUF]
        wsems = bufs_sems[2 * _NBUF :]
        wid = lax.axis_index("s") * _NC + lax.axis_index("c")
        base = wid * (n_chunks * _CHUNK)
        pltpu.sync_copy(idx_hbm.at[wid], idx_v)

        def gather(j, b):
            pltpu.async_copy(table_hbm.at[idx_v.at[j]], bufs[b], gsems[b])

        def wait_gather(j, b):
            pltpu.make_async_copy(table_hbm.at[idx_v.at[j]], bufs[b], gsems[b]).wait()

        def writeback(j, b):
            pltpu.async_copy(bufs[b], out_hbm.at[pl.ds(base + j * _CHUNK, _CHUNK)], wsems[b])

        def wait_writeback(j, b):
            pltpu.make_async_copy(
                bufs[b], out_hbm.at[pl.ds(base + j * _CHUNK, _CHUNK)], wsems[b]
            ).wait()

        for b in range(_NBUF):
            gather(b, b)

        n_rounds = n_chunks // _NBUF

        def body(i, _):
            j0 = i * _NBUF
            for b in range(_NBUF):
                wait_gather(j0 + b, b)
                writeback(j0 + b, b)
            for b in range(_NBUF):
                wait_writeback(j0 + b, b)

                @pl.when(i + 1 < n_rounds)
                def _():
                    gather(j0 + _NBUF + b, b)

            return 0

        lax.fori_loop(0, n_rounds, body, 0)

    return k(idx, table)


def kernel(inputs, targets_table, context_table):
    b, h = inputs.shape
    v, d = targets_table.shape
    total = b * h
    assert total % (_NW * _CHUNK) == 0
    n_chunks = total // (_NW * _CHUNK)
    assert n_chunks % _NBUF == 0
    idx = inputs.reshape(_NW, n_chunks, _CHUNK).astype(jnp.int32)
    out = _sc_gather(idx, targets_table, n_chunks, d)
    return out.reshape(b, h, d)


# speedup vs baseline: 5.9328x; 1.7787x over previous
import functools

import jax
import jax.numpy as jnp
from jax import lax
from jax.experimental import pallas as pl
from jax.experimental.pallas import tpu as pltpu
from jax.experimental.pallas import tpu_sc as plsc

_NC = 2
_NS = 16
_NW = _NC * _NS
_NBUF = 8


@functools.partial(jax.jit, static_argnums=(2, 3, 4))
def _sc_gather(idx, table, b, h, embd):
    b_per_w = b // _NW
    mesh = plsc.VectorSubcoreMesh(core_axis_name="c", subcore_axis_name="s")

    @functools.partial(
        pl.kernel,
        mesh=mesh,
        out_type=jax.ShapeDtypeStruct((b, h, embd), jnp.float32),
        scratch_types=[
            pltpu.VMEM((b_per_w, h), jnp.int32),
        ]
        + [pltpu.VMEM((h, embd), jnp.float32) for _ in range(_NBUF)]
        + [pltpu.SemaphoreType.DMA for _ in range(2 * _NBUF)],
    )
    def k(idx_hbm, table_hbm, out_hbm, idx_v, *bufs_sems):
        bufs = bufs_sems[:_NBUF]
        gsems = bufs_sems[_NBUF : 2 * _NBUF]
        wsems = bufs_sems[2 * _NBUF :]
        wid = lax.axis_index("s") * _NC + lax.axis_index("c")
        base = wid * b_per_w
        pltpu.sync_copy(idx_hbm.at[pl.ds(base, b_per_w)], idx_v)

        def gather(j, s):
            pltpu.async_copy(table_hbm.at[idx_v.at[j]], bufs[s], gsems[s])

        def wait_gather(j, s):
            pltpu.make_async_copy(table_hbm.at[idx_v.at[j]], bufs[s], gsems[s]).wait()

        def writeback(j, s):
            pltpu.async_copy(bufs[s], out_hbm.at[base + j], wsems[s])

        def wait_writeback(j, s):
            pltpu.make_async_copy(bufs[s], out_hbm.at[base + j], wsems[s]).wait()

        for s in range(_NBUF):
            gather(s, s)

        n_rounds = b_per_w // _NBUF

        def body(i, _):
            j0 = i * _NBUF
            for s in range(_NBUF):
                wait_gather(j0 + s, s)
                writeback(j0 + s, s)
            for s in range(_NBUF):
                wait_writeback(j0 + s, s)

                @pl.when(i + 1 < n_rounds)
                def _():
                    gather(j0 + _NBUF + s, s)

            return 0

        lax.fori_loop(0, n_rounds, body, 0)

    return k(idx, table)


def kernel(inputs, targets_table, context_table):
    b, h = inputs.shape
    v, d = targets_table.shape
    assert b % (_NW * _NBUF) == 0
    return _sc_gather(inputs.astype(jnp.int32), targets_table, b, h, d)


# speedup vs baseline: 10.3491x; 1.7444x over previous
import functools

import jax
import jax.numpy as jnp
from jax import lax
from jax.experimental import pallas as pl
from jax.experimental.pallas import tpu as pltpu
from jax.experimental.pallas import tpu_sc as plsc

_NC = 2
_NS = 16
_NW = _NC * _NS
_CHUNK = 128
_NBUF = 5


@functools.partial(jax.jit, static_argnums=(2, 3, 4))
def _sc_gather(idx_t, table, b, h, embd):
    b_per_w = b // _NW
    mesh = plsc.VectorSubcoreMesh(core_axis_name="c", subcore_axis_name="s")

    @functools.partial(
        pl.kernel,
        mesh=mesh,
        out_type=jax.ShapeDtypeStruct((h, b, embd), jnp.float32),
        scratch_types=[
            pltpu.VMEM((h, b_per_w), jnp.int32),
        ]
        + [pltpu.VMEM((b_per_w, embd), jnp.float32) for _ in range(_NBUF)]
        + [pltpu.SemaphoreType.DMA for _ in range(2 * _NBUF)],
    )
    def k(idx_hbm, table_hbm, out_hbm, idx_v, *bufs_sems):
        bufs = bufs_sems[:_NBUF]
        gsems = bufs_sems[_NBUF : 2 * _NBUF]
        wsems = bufs_sems[2 * _NBUF :]
        wid = lax.axis_index("s") * _NC + lax.axis_index("c")
        base = wid * b_per_w
        pltpu.sync_copy(idx_hbm.at[:, pl.ds(base, b_per_w)], idx_v)

        def gather(j, s):
            pltpu.async_copy(table_hbm.at[idx_v.at[j]], bufs[s], gsems[s])

        def wait_gather(j, s):
            pltpu.make_async_copy(table_hbm.at[idx_v.at[j]], bufs[s], gsems[s]).wait()

        def writeback(j, s):
            pltpu.async_copy(bufs[s], out_hbm.at[j, pl.ds(base, b_per_w)], wsems[s])

        def wait_writeback(j, s):
            pltpu.make_async_copy(
                bufs[s], out_hbm.at[j, pl.ds(base, b_per_w)], wsems[s]
            ).wait()

        for s in range(_NBUF):
            gather(s, s)

        n_rounds = h // _NBUF

        def body(i, _):
            j0 = i * _NBUF
            for s in range(_NBUF):
                wait_gather(j0 + s, s)
                writeback(j0 + s, s)
            for s in range(_NBUF):
                wait_writeback(j0 + s, s)

                @pl.when(i + 1 < n_rounds)
                def _():
                    gather(j0 + _NBUF + s, s)

            return 0

        lax.fori_loop(0, n_rounds, body, 0)

    return k(idx_t, table)


def kernel(inputs, targets_table, context_table):
    b, h = inputs.shape
    v, d = targets_table.shape
    assert b % (_NW * _CHUNK) == 0 and h % _NBUF == 0
    out_t = _sc_gather(inputs.T.astype(jnp.int32), targets_table, b, h, d)
    return jnp.transpose(out_t, (1, 0, 2))
